# baseline (speedup 1.0000x reference)
import jax
import jax.numpy as jnp
from jax.experimental import pallas as pl
from jax.experimental.pallas import tpu as pltpu

T = 32768
D = 768
E = 8
P = 2
SCALE = 10.0
EPS = 1e-6
BLOCK_T = 2048


def _router_kernel(h_ref, proto_ref, mask_ref, probs_ref, logits_ref):
    hb = h_ref[...]
    w = proto_ref[...]

    wn = w / (jnp.sqrt(jnp.sum(w * w, axis=-1, keepdims=True)) + EPS)
    hn = hb / (jnp.sqrt(jnp.sum(hb * hb, axis=-1, keepdims=True)) + EPS)

    dn = (((1,), (1,)), ((), ()))
    simsT = jax.lax.dot_general(wn, hn, dn, preferred_element_type=jnp.float32)
    s0 = simsT[0:E, :]
    s1 = simsT[E:2 * E, :]

    m = jnp.maximum(s0, s1)
    lse = m + jnp.log(jnp.exp(s0 - m) + jnp.exp(s1 - m))
    logits = SCALE * lse

    mx = jnp.max(logits, axis=0, keepdims=True)
    ex = jnp.exp(logits - mx)
    probs = ex / jnp.sum(ex, axis=0, keepdims=True)

    iota = jax.lax.broadcasted_iota(jnp.int32, logits.shape, 0)
    cand = jnp.where(logits == mx, iota, E)
    first = jnp.min(cand, axis=0, keepdims=True)
    mask = (iota == first).astype(jnp.float32)

    mask_ref[...] = mask
    probs_ref[...] = probs
    logits_ref[...] = logits


def kernel(h, prototypes):
    proto = jnp.transpose(prototypes, (1, 0, 2)).reshape(P * E, D)

    grid = (T // BLOCK_T,)
    mask_f, probs, logits = pl.pallas_call(
        _router_kernel,
        grid=grid,
        in_specs=[
            pl.BlockSpec((BLOCK_T, D), lambda i: (i, 0)),
            pl.BlockSpec((P * E, D), lambda i: (0, 0)),
        ],
        out_specs=[
            pl.BlockSpec((E, BLOCK_T), lambda i: (0, i)),
            pl.BlockSpec((E, BLOCK_T), lambda i: (0, i)),
            pl.BlockSpec((E, BLOCK_T), lambda i: (0, i)),
        ],
        out_shape=[
            jax.ShapeDtypeStruct((E, T), jnp.float32),
            jax.ShapeDtypeStruct((E, T), jnp.float32),
            jax.ShapeDtypeStruct((E, T), jnp.float32),
        ],
        compiler_params=pltpu.CompilerParams(
            dimension_semantics=("parallel",),
        ),
    )(h, proto)

    logits_t = logits.T
    return (mask_f.T.astype(bool), probs.T, logits_t, logits_t)


# speedup vs baseline: 1.0687x; 1.0687x over previous
import jax
import jax.numpy as jnp
from jax.experimental import pallas as pl
from jax.experimental.pallas import tpu as pltpu

T = 32768
D = 768
E = 8
P = 2
SCALE = 10.0
EPS = 1e-6
BLOCK_T = 4096


def _router_kernel(h_ref, proto_ref, mask_ref, probs_ref, logits_ref):
    hb = h_ref[...]
    w = proto_ref[...]

    wn = w / (jnp.sqrt(jnp.sum(w * w, axis=-1, keepdims=True)) + EPS)
    hn = hb / (jnp.sqrt(jnp.sum(hb * hb, axis=-1, keepdims=True)) + EPS)

    dn = (((1,), (1,)), ((), ()))
    simsT = jax.lax.dot_general(wn, hn, dn, preferred_element_type=jnp.float32)
    s0 = simsT[0:E, :]
    s1 = simsT[E:2 * E, :]

    m = jnp.maximum(s0, s1)
    lse = m + jnp.log(jnp.exp(s0 - m) + jnp.exp(s1 - m))
    logits = SCALE * lse

    mx = jnp.max(logits, axis=0, keepdims=True)
    ex = jnp.exp(logits - mx)
    probs = ex / jnp.sum(ex, axis=0, keepdims=True)

    iota = jax.lax.broadcasted_iota(jnp.int32, logits.shape, 0)
    cand = jnp.where(logits == mx, iota, E)
    first = jnp.min(cand, axis=0, keepdims=True)
    mask = (iota == first).astype(jnp.float32)

    mask_ref[...] = mask
    probs_ref[...] = probs
    logits_ref[...] = logits


def kernel(h, prototypes):
    proto = jnp.transpose(prototypes, (1, 0, 2)).reshape(P * E, D)

    grid = (T // BLOCK_T,)
    mask_f, probs, logits = pl.pallas_call(
        _router_kernel,
        grid=grid,
        in_specs=[
            pl.BlockSpec((BLOCK_T, D), lambda i: (i, 0)),
            pl.BlockSpec((P * E, D), lambda i: (0, 0)),
        ],
        out_specs=[
            pl.BlockSpec((E, BLOCK_T), lambda i: (0, i)),
            pl.BlockSpec((E, BLOCK_T), lambda i: (0, i)),
            pl.BlockSpec((E, BLOCK_T), lambda i: (0, i)),
        ],
        out_shape=[
            jax.ShapeDtypeStruct((E, T), jnp.float32),
            jax.ShapeDtypeStruct((E, T), jnp.float32),
            jax.ShapeDtypeStruct((E, T), jnp.float32),
        ],
        compiler_params=pltpu.CompilerParams(
            dimension_semantics=("parallel",),
        ),
    )(h, proto)

    logits_t = logits.T
    return (mask_f.T.astype(bool), probs.T, logits_t, logits_t)


# speedup vs baseline: 1.0856x; 1.0158x over previous
import jax
import jax.numpy as jnp
from jax.experimental import pallas as pl
from jax.experimental.pallas import tpu as pltpu

T = 32768
D = 768
E = 8
P = 2
SCALE = 10.0
EPS = 1e-6
BLOCK_T = 4096


def _router_kernel(h_ref, proto_ref, mask_ref, probs_ref, logits_ref):
    hb = h_ref[...]
    w = proto_ref[...]

    wn = w / (jnp.sqrt(jnp.sum(w * w, axis=-1, keepdims=True)) + EPS)
    hn = hb / (jnp.sqrt(jnp.sum(hb * hb, axis=-1, keepdims=True)) + EPS)

    dn = (((1,), (1,)), ((), ()))
    simsT = jax.lax.dot_general(wn, hn, dn, preferred_element_type=jnp.float32)
    s0 = simsT[0:E, :]
    s1 = simsT[E:2 * E, :]

    m = jnp.maximum(s0, s1)
    lse = m + jnp.log(jnp.exp(s0 - m) + jnp.exp(s1 - m))
    logits = SCALE * lse

    mx = jnp.max(logits, axis=0, keepdims=True)
    ex = jnp.exp(logits - mx)
    probs = ex / jnp.sum(ex, axis=0, keepdims=True)

    iota = jax.lax.broadcasted_iota(jnp.int32, logits.shape, 0)
    cand = jnp.where(logits == mx, iota, E)
    first = jnp.min(cand, axis=0, keepdims=True)
    mask = (iota == first).astype(jnp.int8)

    mask_ref[...] = mask
    probs_ref[...] = probs
    logits_ref[...] = logits


def kernel(h, prototypes):
    proto = jnp.transpose(prototypes, (1, 0, 2)).reshape(P * E, D)

    grid = (T // BLOCK_T,)
    mask_f, probs, logits = pl.pallas_call(
        _router_kernel,
        grid=grid,
        in_specs=[
            pl.BlockSpec((BLOCK_T, D), lambda i: (i, 0)),
            pl.BlockSpec((P * E, D), lambda i: (0, 0)),
        ],
        out_specs=[
            pl.BlockSpec((E, BLOCK_T), lambda i: (0, i)),
            pl.BlockSpec((E, BLOCK_T), lambda i: (0, i)),
            pl.BlockSpec((E, BLOCK_T), lambda i: (0, i)),
        ],
        out_shape=[
            jax.ShapeDtypeStruct((E, T), jnp.int8),
            jax.ShapeDtypeStruct((E, T), jnp.float32),
            jax.ShapeDtypeStruct((E, T), jnp.float32),
        ],
        compiler_params=pltpu.CompilerParams(
            dimension_semantics=("parallel",),
        ),
    )(h, proto)

    logits_t = logits.T
    return (mask_f.T.astype(bool), probs.T, logits_t, logits_t)
